# baseline (speedup 1.0000x reference)
import jax
import jax.numpy as jnp
from jax import lax
from jax.experimental import pallas as pl
from jax.experimental.pallas import tpu as pltpu


def _fused_kernel(prem_ref, hyp_ref,
                  emb_ref,
                  w_ih_ref, w_hh_ref, b_ref,
                  w1_ref, b1_ref,
                  w2_ref, b2_ref,
                  w3_ref, b3_ref,
                  out_ref,
                  x_buf,
                  w_ih_v, w_hh_v, b_v, w1_v, b1_v, w2_v, b2_v, w3_v, b3_v,
                  g_sem, w_sem):
    B, S = prem_ref.shape
    E = w_ih_ref.shape[0]
    H = w_hh_ref.shape[0]
    B2 = 2 * B
    M = S * B2
    H4 = 4 * H
    TC = 4
    bounds = []
    t_hi = S
    while t_hi > 0:
        t_lo = max(t_hi - TC, 0)
        bounds.append((t_lo, t_hi))
        t_hi = t_lo
    NC = len(bounds)

    def row_copy(t, r, sem):
        tok = prem_ref[r, t] if r < B else hyp_ref[r - B, t]
        return pltpu.make_async_copy(emb_ref.at[pl.ds(tok, 1), :],
                                     x_buf.at[t * B2 + r], sem)

    for t in range(bounds[0][1] - 1, bounds[0][0] - 1, -1):
        for r in range(B2):
            row_copy(t, r, g_sem.at[0]).start()

    w_pairs = ((w_ih_ref, w_ih_v), (w_hh_ref, w_hh_v), (b_ref, b_v),
               (w1_ref, w1_v), (b1_ref, b1_v), (w2_ref, w2_v),
               (b2_ref, b2_v), (w3_ref, w3_v), (b3_ref, b3_v))
    for src, dst in w_pairs:
        pltpu.make_async_copy(src, dst, w_sem).start()

    for ci in range(1, NC):
        for t in range(bounds[ci][1] - 1, bounds[ci][0] - 1, -1):
            for r in range(B2):
                row_copy(t, r, g_sem.at[ci]).start()

    for src, dst in w_pairs:
        pltpu.make_async_copy(src, dst, w_sem).wait()

    gate_q = lax.broadcasted_iota(jnp.int32, (1, H4), 1) // H
    gscale = jnp.where(gate_q == 2, 1.0, 0.5).astype(jnp.float32)
    w_hh_s = w_hh_v[...] * gscale
    b_s = b_v[...] * gscale
    w_ih_s = w_ih_v[...] * gscale

    def gates(z):
        th = jnp.tanh(z)
        return (th[:, :H], th[:, H:2 * H], th[:, 2 * H:3 * H], th[:, 3 * H:])

    h = None
    c = None
    for ci in range(NC):
        t_lo, t_hi = bounds[ci]
        base = t_lo * B2
        CR = (t_hi - t_lo) * B2
        pltpu.make_async_copy(emb_ref.at[pl.ds(0, CR), :],
                              x_buf.at[pl.ds(base, CR), 0],
                              g_sem.at[ci]).wait()
        xc = x_buf[pl.ds(base, CR), 0, :]
        gxc = (jnp.dot(xc, w_ih_s, preferred_element_type=jnp.float32)
               + b_s)
        for t in range(t_hi - 1, t_lo - 1, -1):
            zt = gxc[(t - t_lo) * B2:(t - t_lo + 1) * B2, :]
            if h is None:
                i_g, _, g_g, o_g = gates(zt)
                c = (0.5 * i_g + 0.5) * g_g
            else:
                z = zt + jnp.dot(h, w_hh_s,
                                 preferred_element_type=jnp.float32)
                i_g, f_g, g_g, o_g = gates(z)
                c = (0.5 * f_g + 0.5) * c + (0.5 * i_g + 0.5) * g_g
            h = (0.5 * o_g + 0.5) * jnp.tanh(c)

    y = jnp.maximum(
        jnp.dot(h[:B, :], w1_v[:H, :], preferred_element_type=jnp.float32)
        + jnp.dot(h[B:, :], w1_v[H:, :], preferred_element_type=jnp.float32)
        + b1_v[...], 0.0)
    y = jnp.maximum(
        jnp.dot(y, w2_v[...], preferred_element_type=jnp.float32)
        + b2_v[...], 0.0)
    y = jnp.maximum(
        jnp.dot(y, w3_v[...], preferred_element_type=jnp.float32)
        + b3_v[...], 0.0)
    out_ref[...] = y.astype(out_ref.dtype)


@jax.jit
def _forward(embedding, w_ih_rev, w_hh_rev, b_lstm_rev,
             w1, b1, w2, b2, w3, b3, premise, hypothesis):
    B, S = premise.shape
    V, E = embedding.shape
    H = w_hh_rev.shape[0]
    C = w3.shape[1]
    M = S * 2 * B

    dense = (w_ih_rev, w_hh_rev, b_lstm_rev, w1, b1, w2, b2, w3, b3)

    smem = pl.BlockSpec(memory_space=pltpu.MemorySpace.SMEM)
    hbm = pl.BlockSpec(memory_space=pltpu.MemorySpace.HBM)
    return pl.pallas_call(
        _fused_kernel,
        out_shape=jax.ShapeDtypeStruct((B, C), jnp.float32),
        grid=(1,),
        in_specs=[smem, smem] + [hbm] * 10,
        out_specs=pl.BlockSpec((B, C), lambda i: (0, 0)),
        scratch_shapes=[pltpu.VMEM((M, 1, E), jnp.float32)]
                       + [pltpu.VMEM(a.shape, jnp.float32) for a in dense]
                       + [pltpu.SemaphoreType.DMA(((S + 3) // 4,)),
                          pltpu.SemaphoreType.DMA],
        compiler_params=pltpu.CompilerParams(
            dimension_semantics=("arbitrary",)),
    )(premise, hypothesis, embedding, *dense)


def kernel(embedding, w_ih_rev, w_hh_rev, b_lstm_rev,
           w1, b1, w2, b2, w3, b3, premise, hypothesis):
    return _forward(embedding, w_ih_rev, w_hh_rev, b_lstm_rev,
                    w1, b1, w2, b2, w3, b3, premise, hypothesis)


# speedup vs baseline: 1.0468x; 1.0468x over previous
import jax
import jax.numpy as jnp
from jax import lax
from jax.experimental import pallas as pl
from jax.experimental.pallas import tpu as pltpu


def _fused_kernel(prem_ref, hyp_ref,
                  emb_ref,
                  w_ih_ref, w_hh_ref, b_ref,
                  w1_ref, b1_ref,
                  w2_ref, b2_ref,
                  w3_ref, b3_ref,
                  out_ref,
                  x_buf, dma_sem):
    B, S = prem_ref.shape
    E = w_ih_ref.shape[0]
    H = w_hh_ref.shape[0]
    B2 = 2 * B
    M = S * B2
    H4 = 4 * H

    for t in range(S - 1, -1, -1):
        for r in range(B2):
            tok = prem_ref[r, t] if r < B else hyp_ref[r - B, t]
            pltpu.make_async_copy(emb_ref.at[pl.ds(tok, 1), :],
                                  x_buf.at[t * B2 + r], dma_sem).start()

    gate_q = lax.broadcasted_iota(jnp.int32, (1, H4), 1) // H
    gscale = jnp.where(gate_q == 2, 1.0, 0.5).astype(jnp.float32)
    w_hh_s = w_hh_ref[...] * gscale
    b_s = b_ref[...] * gscale
    w_ih_s = w_ih_ref[...] * gscale

    pltpu.make_async_copy(emb_ref.at[pl.ds(0, M), :],
                          x_buf.at[pl.ds(0, M), 0], dma_sem).wait()

    x = x_buf[:, 0, :]
    gx = (jnp.dot(x, w_ih_s, preferred_element_type=jnp.float32)
          + b_s)

    def gates(z):
        th = jnp.tanh(z)
        return (th[:, :H], th[:, H:2 * H], th[:, 2 * H:3 * H], th[:, 3 * H:])

    i_g, _, g_g, o_g = gates(gx[(S - 1) * B2:S * B2, :])
    c = (0.5 * i_g + 0.5) * g_g
    h = (0.5 * o_g + 0.5) * jnp.tanh(c)
    for t in range(S - 2, -1, -1):
        z = gx[t * B2:(t + 1) * B2, :] + jnp.dot(
            h, w_hh_s, preferred_element_type=jnp.float32)
        i_g, f_g, g_g, o_g = gates(z)
        c = (0.5 * f_g + 0.5) * c + (0.5 * i_g + 0.5) * g_g
        h = (0.5 * o_g + 0.5) * jnp.tanh(c)

    y = jnp.maximum(
        jnp.dot(h[:B, :], w1_ref[:H, :], preferred_element_type=jnp.float32)
        + jnp.dot(h[B:, :], w1_ref[H:, :], preferred_element_type=jnp.float32)
        + b1_ref[...], 0.0)
    y = jnp.maximum(
        jnp.dot(y, w2_ref[...], preferred_element_type=jnp.float32)
        + b2_ref[...], 0.0)
    y = jnp.maximum(
        jnp.dot(y, w3_ref[...], preferred_element_type=jnp.float32)
        + b3_ref[...], 0.0)
    out_ref[...] = y.astype(out_ref.dtype)


@jax.jit
def _forward(embedding, w_ih_rev, w_hh_rev, b_lstm_rev,
             w1, b1, w2, b2, w3, b3, premise, hypothesis):
    B, S = premise.shape
    V, E = embedding.shape
    C = w3.shape[1]
    M = S * 2 * B

    dense = (w_ih_rev, w_hh_rev, b_lstm_rev, w1, b1, w2, b2, w3, b3)

    def vmem_spec(a):
        nd = a.ndim
        return pl.BlockSpec(a.shape, lambda i, nd=nd: (0,) * nd)

    smem = pl.BlockSpec(memory_space=pltpu.MemorySpace.SMEM)
    return pl.pallas_call(
        _fused_kernel,
        out_shape=jax.ShapeDtypeStruct((B, C), jnp.float32),
        grid=(1,),
        in_specs=[smem, smem,
                  pl.BlockSpec(memory_space=pltpu.MemorySpace.HBM)]
                 + [vmem_spec(a) for a in dense],
        out_specs=pl.BlockSpec((B, C), lambda i: (0, 0)),
        scratch_shapes=[pltpu.VMEM((M, 1, E), jnp.float32),
                        pltpu.SemaphoreType.DMA],
        compiler_params=pltpu.CompilerParams(
            dimension_semantics=("arbitrary",)),
    )(premise, hypothesis, embedding, *dense)


def kernel(embedding, w_ih_rev, w_hh_rev, b_lstm_rev,
           w1, b1, w2, b2, w3, b3, premise, hypothesis):
    return _forward(embedding, w_ih_rev, w_hh_rev, b_lstm_rev,
                    w1, b1, w2, b2, w3, b3, premise, hypothesis)


# speedup vs baseline: 1.0528x; 1.0057x over previous
import jax
import jax.numpy as jnp
from jax import lax
from jax.experimental import pallas as pl
from jax.experimental.pallas import tpu as pltpu


def _fused_kernel(prem_ref, hyp_ref,
                  emb_ref,
                  w_ih_ref, w_hh_ref, b_ref,
                  w1_ref, b1_ref,
                  w2_ref, b2_ref,
                  w3_ref, b3_ref,
                  out_ref,
                  x_buf, dma_sem):
    B, S = prem_ref.shape
    E = w_ih_ref.shape[0]
    H = w_hh_ref.shape[0]
    B2 = 2 * B
    M = S * B2
    H4 = 4 * H

    NQ = 4
    mi = 0
    for t in range(S - 1, -1, -1):
        for r in range(B2):
            tok = prem_ref[r, t] if r < B else hyp_ref[r - B, t]
            pltpu.make_async_copy(emb_ref.at[pl.ds(tok, 1), :],
                                  x_buf.at[t * B2 + r],
                                  dma_sem.at[mi % NQ]).start()
            mi += 1

    gate_q = lax.broadcasted_iota(jnp.int32, (1, H4), 1) // H
    gscale = jnp.where(gate_q == 2, 1.0, 0.5).astype(jnp.float32)
    w_hh_s = w_hh_ref[...] * gscale
    b_s = b_ref[...] * gscale
    w_ih_s = w_ih_ref[...] * gscale

    for q in range(NQ):
        nq = M // NQ + (1 if q < M % NQ else 0)
        pltpu.make_async_copy(emb_ref.at[pl.ds(0, nq), :],
                              x_buf.at[pl.ds(0, nq), 0],
                              dma_sem.at[q]).wait()

    x = x_buf[:, 0, :]
    gx = (jnp.dot(x, w_ih_s, preferred_element_type=jnp.float32)
          + b_s)

    def gates(z):
        th = jnp.tanh(z)
        return (th[:, :H], th[:, H:2 * H], th[:, 2 * H:3 * H], th[:, 3 * H:])

    i_g, _, g_g, o_g = gates(gx[(S - 1) * B2:S * B2, :])
    c = (0.5 * i_g + 0.5) * g_g
    h = (0.5 * o_g + 0.5) * jnp.tanh(c)
    for t in range(S - 2, -1, -1):
        z = gx[t * B2:(t + 1) * B2, :] + jnp.dot(
            h, w_hh_s, preferred_element_type=jnp.float32)
        i_g, f_g, g_g, o_g = gates(z)
        c = (0.5 * f_g + 0.5) * c + (0.5 * i_g + 0.5) * g_g
        h = (0.5 * o_g + 0.5) * jnp.tanh(c)

    y = jnp.maximum(
        jnp.dot(h[:B, :], w1_ref[:H, :], preferred_element_type=jnp.float32)
        + jnp.dot(h[B:, :], w1_ref[H:, :], preferred_element_type=jnp.float32)
        + b1_ref[...], 0.0)
    y = jnp.maximum(
        jnp.dot(y, w2_ref[...], preferred_element_type=jnp.float32)
        + b2_ref[...], 0.0)
    y = jnp.maximum(
        jnp.dot(y, w3_ref[...], preferred_element_type=jnp.float32)
        + b3_ref[...], 0.0)
    out_ref[...] = y.astype(out_ref.dtype)


@jax.jit
def _forward(embedding, w_ih_rev, w_hh_rev, b_lstm_rev,
             w1, b1, w2, b2, w3, b3, premise, hypothesis):
    B, S = premise.shape
    V, E = embedding.shape
    C = w3.shape[1]
    M = S * 2 * B

    dense = (w_ih_rev, w_hh_rev, b_lstm_rev, w1, b1, w2, b2, w3, b3)

    def vmem_spec(a):
        nd = a.ndim
        return pl.BlockSpec(a.shape, lambda i, nd=nd: (0,) * nd)

    smem = pl.BlockSpec(memory_space=pltpu.MemorySpace.SMEM)
    return pl.pallas_call(
        _fused_kernel,
        out_shape=jax.ShapeDtypeStruct((B, C), jnp.float32),
        grid=(1,),
        in_specs=[smem, smem,
                  pl.BlockSpec(memory_space=pltpu.MemorySpace.HBM)]
                 + [vmem_spec(a) for a in dense],
        out_specs=pl.BlockSpec((B, C), lambda i: (0, 0)),
        scratch_shapes=[pltpu.VMEM((M, 1, E), jnp.float32),
                        pltpu.SemaphoreType.DMA((4,))],
        compiler_params=pltpu.CompilerParams(
            dimension_semantics=("arbitrary",)),
    )(premise, hypothesis, embedding, *dense)


def kernel(embedding, w_ih_rev, w_hh_rev, b_lstm_rev,
           w1, b1, w2, b2, w3, b3, premise, hypothesis):
    return _forward(embedding, w_ih_rev, w_hh_rev, b_lstm_rev,
                    w1, b1, w2, b2, w3, b3, premise, hypothesis)


# speedup vs baseline: 1.0667x; 1.0132x over previous
import jax
import jax.numpy as jnp
from jax import lax
from jax.experimental import pallas as pl
from jax.experimental.pallas import tpu as pltpu


def _fused_kernel(prem_ref, hyp_ref,
                  emb_ref,
                  w_ih_ref, w_hh_ref, b_ref,
                  w1_ref, b1_ref,
                  w2_ref, b2_ref,
                  w3_ref, b3_ref,
                  out_ref,
                  x_buf,
                  w_ih_v, w_hh_v, b_v, w1_v, b1_v, w2_v, b2_v, w3_v, b3_v,
                  g_sem, s0, s1, s2):
    B, S = prem_ref.shape
    E = w_ih_ref.shape[0]
    H = w_hh_ref.shape[0]
    B2 = 2 * B
    M = S * B2
    H4 = 4 * H

    pltpu.make_async_copy(w_ih_ref, w_ih_v, s0).start()
    pltpu.make_async_copy(b_ref, b_v, s0).start()

    for t in range(S - 1, -1, -1):
        for r in range(B2):
            tok = prem_ref[r, t] if r < B else hyp_ref[r - B, t]
            pltpu.make_async_copy(emb_ref.at[pl.ds(tok, 1), :],
                                  x_buf.at[t * B2 + r], g_sem).start()

    pltpu.make_async_copy(w_hh_ref, w_hh_v, s1).start()
    for src, dst in ((w1_ref, w1_v), (b1_ref, b1_v), (w2_ref, w2_v),
                     (b2_ref, b2_v), (w3_ref, w3_v), (b3_ref, b3_v)):
        pltpu.make_async_copy(src, dst, s2).start()

    gate_q = lax.broadcasted_iota(jnp.int32, (1, H4), 1) // H
    gscale = jnp.where(gate_q == 2, 1.0, 0.5).astype(jnp.float32)
    pltpu.make_async_copy(w_ih_ref, w_ih_v, s0).wait()
    pltpu.make_async_copy(b_ref, b_v, s0).wait()
    w_ih_s = w_ih_v[...] * gscale
    b_s = b_v[...] * gscale

    pltpu.make_async_copy(emb_ref.at[pl.ds(0, M), :],
                          x_buf.at[pl.ds(0, M), 0], g_sem).wait()

    x = x_buf[:, 0, :]
    gx = (jnp.dot(x, w_ih_s, preferred_element_type=jnp.float32)
          + b_s)

    def gates(z):
        th = jnp.tanh(z)
        return (th[:, :H], th[:, H:2 * H], th[:, 2 * H:3 * H], th[:, 3 * H:])

    i_g, _, g_g, o_g = gates(gx[(S - 1) * B2:S * B2, :])
    c = (0.5 * i_g + 0.5) * g_g
    h = (0.5 * o_g + 0.5) * jnp.tanh(c)

    pltpu.make_async_copy(w_hh_ref, w_hh_v, s1).wait()
    w_hh_s = w_hh_v[...] * gscale
    for t in range(S - 2, -1, -1):
        z = gx[t * B2:(t + 1) * B2, :] + jnp.dot(
            h, w_hh_s, preferred_element_type=jnp.float32)
        i_g, f_g, g_g, o_g = gates(z)
        c = (0.5 * f_g + 0.5) * c + (0.5 * i_g + 0.5) * g_g
        h = (0.5 * o_g + 0.5) * jnp.tanh(c)

    for src, dst in ((w1_ref, w1_v), (b1_ref, b1_v), (w2_ref, w2_v),
                     (b2_ref, b2_v), (w3_ref, w3_v), (b3_ref, b3_v)):
        pltpu.make_async_copy(src, dst, s2).wait()

    y = jnp.maximum(
        jnp.dot(h[:B, :], w1_v[:H, :], preferred_element_type=jnp.float32)
        + jnp.dot(h[B:, :], w1_v[H:, :], preferred_element_type=jnp.float32)
        + b1_v[...], 0.0)
    y = jnp.maximum(
        jnp.dot(y, w2_v[...], preferred_element_type=jnp.float32)
        + b2_v[...], 0.0)
    y = jnp.maximum(
        jnp.dot(y, w3_v[...], preferred_element_type=jnp.float32)
        + b3_v[...], 0.0)
    out_ref[...] = y.astype(out_ref.dtype)


@jax.jit
def _forward(embedding, w_ih_rev, w_hh_rev, b_lstm_rev,
             w1, b1, w2, b2, w3, b3, premise, hypothesis):
    B, S = premise.shape
    V, E = embedding.shape
    C = w3.shape[1]
    M = S * 2 * B

    dense = (w_ih_rev, w_hh_rev, b_lstm_rev, w1, b1, w2, b2, w3, b3)

    smem = pl.BlockSpec(memory_space=pltpu.MemorySpace.SMEM)
    hbm = pl.BlockSpec(memory_space=pltpu.MemorySpace.HBM)
    return pl.pallas_call(
        _fused_kernel,
        out_shape=jax.ShapeDtypeStruct((B, C), jnp.float32),
        grid=(1,),
        in_specs=[smem, smem] + [hbm] * 10,
        out_specs=pl.BlockSpec((B, C), lambda i: (0, 0)),
        scratch_shapes=[pltpu.VMEM((M, 1, E), jnp.float32)]
                       + [pltpu.VMEM(a.shape, jnp.float32) for a in dense]
                       + [pltpu.SemaphoreType.DMA] * 4,
        compiler_params=pltpu.CompilerParams(
            dimension_semantics=("arbitrary",)),
    )(premise, hypothesis, embedding, *dense)


def kernel(embedding, w_ih_rev, w_hh_rev, b_lstm_rev,
           w1, b1, w2, b2, w3, b3, premise, hypothesis):
    return _forward(embedding, w_ih_rev, w_hh_rev, b_lstm_rev,
                    w1, b1, w2, b2, w3, b3, premise, hypothesis)
